# Initial kernel scaffold; baseline (speedup 1.0000x reference)
import jax
import jax.numpy as jnp
from jax.experimental import pallas as pl


def kernel(positions, hash_table):
    raise NotImplementedError("write your pallas kernel here")


# speedup vs baseline: 1.2777x; 1.2777x over previous
import functools

import numpy as np
import jax
import jax.numpy as jnp
from jax import lax
from jax.experimental import pallas as pl
from jax.experimental.pallas import tpu as pltpu
from jax.experimental.pallas import tpu_sc as plsc

_N_LEVELS = 16
_BASE_RES = 16
_MAX_PARAMS = 2 ** 19
_B_SCALE = 1.3195079565048218
_P1 = int(np.uint32(2654435761).view(np.int32))
_P2 = int(np.uint32(805459861).view(np.int32))
_HASH_MASK = _MAX_PARAMS - 1


def _levels():
    out = []
    off = 0
    for i in range(_N_LEVELS):
        scale = _BASE_RES * np.exp(i * np.log(_B_SCALE)) - 1.0
        res = int(np.ceil(scale)) + 1
        params = res ** 3
        if params % 8 != 0:
            params = ((params + 7) // 8) * 8
        params = min(_MAX_PARAMS, params)
        dense = res ** 3 <= params
        out.append((np.float32(scale), res, params, off, dense))
        off += params
    return out, off


_LEVEL_META, _TOTAL_ROWS = _levels()

_NW = 32
_B = 512
_G = _B // 16


def _encode_body(px_hbm, py_hbm, pz_hbm, tab_hbm, out_hbm,
                 xs_v, ys_v, zs_v, idx0_v, idx1_v, w_v, rows0_v, rows1_v,
                 out_v, sem):
    wid = lax.axis_index("s") * 2 + lax.axis_index("c")
    n = px_hbm.shape[0]
    npw = n // _NW
    nchunks = npw // _B
    lane = lax.iota(jnp.int32, 16)

    def chunk_body(i, _):
        base = wid * npw + i * _B
        pltpu.sync_copy(px_hbm.at[pl.ds(base, _B)], xs_v)
        pltpu.sync_copy(py_hbm.at[pl.ds(base, _B)], ys_v)
        pltpu.sync_copy(pz_hbm.at[pl.ds(base, _B)], zs_v)

        for level in range(_N_LEVELS):
            scale, res, size, off, dense = _LEVEL_META[level]

            def idx_body(g, _, scale=scale, res=res, size=size, off=off,
                         dense=dense):
                s = g * 16
                x = xs_v[pl.ds(s, 16)]
                y = ys_v[pl.ds(s, 16)]
                z = zs_v[pl.ds(s, 16)]
                fscale = jnp.float32(scale)
                pxv = x * fscale + jnp.float32(0.5)
                pyv = y * fscale + jnp.float32(0.5)
                pzv = z * fscale + jnp.float32(0.5)
                ix = pxv.astype(jnp.int32)
                iy = pyv.astype(jnp.int32)
                iz = pzv.astype(jnp.int32)
                fx = pxv - ix.astype(jnp.float32)
                fy = pyv - iy.astype(jnp.float32)
                fz = pzv - iz.astype(jnp.float32)
                wx = (jnp.float32(1.0) - fx, fx)
                wy = (jnp.float32(1.0) - fy, fy)
                wz = (jnp.float32(1.0) - fz, fz)
                if dense:
                    res2 = res * res
                    cx = (ix, ix + 1)
                    ty = (iy * res, iy * res + res)
                    tz = (iz * res2, iz * res2 + res2)
                else:
                    cx = (ix, ix + 1)
                    ty = (iy * _P1, iy * _P1 + _P1)
                    tz = (iz * _P2, iz * _P2 + _P2)
                for corner in range(8):
                    bx, by, bz = corner & 1, (corner >> 1) & 1, (corner >> 2) & 1
                    if dense:
                        h = cx[bx] + ty[by] + tz[bz]
                        hidx = jnp.where(h >= size, h - size, h)
                    else:
                        h = cx[bx] ^ ty[by] ^ tz[bz]
                        hidx = h & _HASH_MASK
                    el0 = (hidx + off) * 2
                    w = wx[bx] * wy[by] * wz[bz]
                    idx0_v[pl.ds(corner * _B + s, 16)] = el0
                    idx1_v[pl.ds(corner * _B + s, 16)] = el0 + 1
                    w_v[pl.ds(corner * _B + s, 16)] = w
                return 0

            lax.fori_loop(0, _G, idx_body, 0, unroll=False)
            cp0 = pltpu.async_copy(tab_hbm.at[idx0_v], rows0_v, sem)
            cp1 = pltpu.async_copy(tab_hbm.at[idx1_v], rows1_v, sem)
            cp0.wait()
            cp1.wait()

            def red_body(g, _, level=level):
                s = g * 16
                acc0 = jnp.zeros((16,), jnp.float32)
                acc1 = jnp.zeros((16,), jnp.float32)
                for corner in range(8):
                    f0 = rows0_v[pl.ds(corner * _B + s, 16)]
                    f1 = rows1_v[pl.ds(corner * _B + s, 16)]
                    w = w_v[pl.ds(corner * _B + s, 16)]
                    acc0 = acc0 + w * f0
                    acc1 = acc1 + w * f1
                oidx = (s + lane) * 32 + (2 * level)
                plsc.store_scatter(out_v, [oidx], acc0)
                plsc.store_scatter(out_v, [oidx + 1], acc1)
                return 0

            lax.fori_loop(0, _G, red_body, 0, unroll=False)

        pltpu.sync_copy(out_v, out_hbm.at[pl.ds(base * 32, _B * 32)])
        return 0

    lax.fori_loop(0, nchunks, chunk_body, 0, unroll=False)


def kernel(positions, hash_table):
    n = positions.shape[0]
    px = positions[:, 0]
    py = positions[:, 1]
    pz = positions[:, 2]
    tab = hash_table

    mesh = plsc.VectorSubcoreMesh(core_axis_name="c", subcore_axis_name="s")
    run = functools.partial(
        pl.kernel,
        mesh=mesh,
        compiler_params=pltpu.CompilerParams(needs_layout_passes=False),
        out_type=jax.ShapeDtypeStruct((n * 32,), jnp.float32),
        scratch_types=[
            pltpu.VMEM((_B,), jnp.float32),
            pltpu.VMEM((_B,), jnp.float32),
            pltpu.VMEM((_B,), jnp.float32),
            pltpu.VMEM((8 * _B,), jnp.int32),
            pltpu.VMEM((8 * _B,), jnp.int32),
            pltpu.VMEM((8 * _B,), jnp.float32),
            pltpu.VMEM((8 * _B,), jnp.float32),
            pltpu.VMEM((8 * _B,), jnp.float32),
            pltpu.VMEM((_B * 32,), jnp.float32),
            pltpu.SemaphoreType.DMA,
        ],
    )(_encode_body)
    out = run(px, py, pz, tab)
    return out.reshape(n, 32)


# speedup vs baseline: 2.4364x; 1.9068x over previous
import functools

import numpy as np
import jax
import jax.numpy as jnp
from jax import lax
from jax.experimental import pallas as pl
from jax.experimental.pallas import tpu as pltpu
from jax.experimental.pallas import tpu_sc as plsc

_N_LEVELS = 16
_BASE_RES = 16
_MAX_PARAMS = 2 ** 19
_B_SCALE = 1.3195079565048218
_P1 = int(np.uint32(2654435761).view(np.int32))
_P2 = int(np.uint32(805459861).view(np.int32))
_HASH_MASK = _MAX_PARAMS - 1


def _levels():
    out = []
    off = 0
    for i in range(_N_LEVELS):
        scale = _BASE_RES * np.exp(i * np.log(_B_SCALE)) - 1.0
        res = int(np.ceil(scale)) + 1
        params = res ** 3
        if params % 8 != 0:
            params = ((params + 7) // 8) * 8
        params = min(_MAX_PARAMS, params)
        dense = res ** 3 <= params
        out.append((np.float32(scale), res, params, off, dense))
        off += params
    return out, off


_LEVEL_META, _TOTAL_ROWS = _levels()

_NW = 32
_B = 512
_G = _B // 16

_N_DENSE = sum(1 for m in _LEVEL_META if m[4])
_DENSE_FLOATS = 2 * sum(m[2] for m in _LEVEL_META[:_N_DENSE])
_STAGE_CHUNK = 8 * _B
_STAGE_ITERS = -(-_DENSE_FLOATS // _STAGE_CHUNK)
_DENSE_PAD = _STAGE_ITERS * _STAGE_CHUNK


def _encode_body(px_hbm, py_hbm, pz_hbm, tab_hbm, out_hbm,
                 xs_v, ys_v, zs_v,
                 idx0_a, idx1_a, idx0_b, idx1_b,
                 w_a, w_b, rows0_a, rows1_a, rows0_b, rows1_b,
                 out_v, tab_s, sem_a, sem_b):
    wid = lax.axis_index("s") * 2 + lax.axis_index("c")
    sid = lax.axis_index("s")

    def stage_body(k, _):
        @pl.when(lax.rem(k, 16) == sid)
        def _():
            pltpu.sync_copy(tab_hbm.at[pl.ds(k * _STAGE_CHUNK, _STAGE_CHUNK)],
                            rows0_a)
            pltpu.sync_copy(rows0_a,
                            tab_s.at[pl.ds(k * _STAGE_CHUNK, _STAGE_CHUNK)])
        return 0

    lax.fori_loop(0, _STAGE_ITERS, stage_body, 0, unroll=False)
    plsc.subcore_barrier()
    n = px_hbm.shape[0]
    npw = n // _NW
    nchunks = npw // _B
    lane = lax.iota(jnp.int32, 16)
    idx_bufs = ((idx0_a, idx1_a), (idx0_b, idx1_b))
    w_bufs = (w_a, w_b)
    row_bufs = ((rows0_a, rows1_a), (rows0_b, rows1_b))
    sems = (sem_a, sem_b)

    def chunk_body(i, _):
        base = wid * npw + i * _B
        pltpu.sync_copy(px_hbm.at[pl.ds(base, _B)], xs_v)
        pltpu.sync_copy(py_hbm.at[pl.ds(base, _B)], ys_v)
        pltpu.sync_copy(pz_hbm.at[pl.ds(base, _B)], zs_v)

        def make_idx_body(level):
            scale, res, size, off, dense = _LEVEL_META[level]
            idx0_v, idx1_v = idx_bufs[level % 2]
            w_v = w_bufs[level % 2]

            def idx_body(g, _):
                s = g * 16
                x = xs_v[pl.ds(s, 16)]
                y = ys_v[pl.ds(s, 16)]
                z = zs_v[pl.ds(s, 16)]
                fscale = jnp.float32(scale)
                pxv = x * fscale + jnp.float32(0.5)
                pyv = y * fscale + jnp.float32(0.5)
                pzv = z * fscale + jnp.float32(0.5)
                ix = pxv.astype(jnp.int32)
                iy = pyv.astype(jnp.int32)
                iz = pzv.astype(jnp.int32)
                fx = pxv - ix.astype(jnp.float32)
                fy = pyv - iy.astype(jnp.float32)
                fz = pzv - iz.astype(jnp.float32)
                wx = (jnp.float32(1.0) - fx, fx)
                wy = (jnp.float32(1.0) - fy, fy)
                wz = (jnp.float32(1.0) - fz, fz)
                if dense:
                    res2 = res * res
                    cx = (ix + off, ix + off + 1)
                    ty = (iy * res, iy * res + res)
                    tz = (iz * res2, iz * res2 + res2)
                    lim = size + off
                else:
                    cx = (ix, ix + 1)
                    ty = (iy * _P1, iy * _P1 + _P1)
                    tz = (iz * _P2, iz * _P2 + _P2)
                for corner in range(8):
                    bx, by, bz = corner & 1, (corner >> 1) & 1, (corner >> 2) & 1
                    if dense:
                        h = cx[bx] + ty[by] + tz[bz]
                        row = jnp.where(h >= lim, h - size, h)
                    else:
                        h = cx[bx] ^ ty[by] ^ tz[bz]
                        row = (h & _HASH_MASK) + off
                    el0 = row * 2
                    w = wx[bx] * wy[by] * wz[bz]
                    idx0_v[pl.ds(corner * _B + s, 16)] = el0
                    idx1_v[pl.ds(corner * _B + s, 16)] = el0 + 1
                    w_v[pl.ds(corner * _B + s, 16)] = w
                return 0

            return idx_body

        def run_idx(level):
            body = make_idx_body(level)

            def _b(g):
                body(g, 0)

            plsc.parallel_loop(0, _G, 1, unroll=2)(_b)

        def issue(level):
            b = level % 2
            src = tab_s if level < _N_DENSE else tab_hbm
            cp0 = pltpu.async_copy(src.at[idx_bufs[b][0]],
                                   row_bufs[b][0], sems[b])
            cp1 = pltpu.async_copy(src.at[idx_bufs[b][1]],
                                   row_bufs[b][1], sems[b])
            return (cp0, cp1)

        def make_red_body(level):
            rows0_v, rows1_v = row_bufs[level % 2]
            w_v = w_bufs[level % 2]

            def red_body(g, _):
                s = g * 16
                acc0 = jnp.zeros((16,), jnp.float32)
                acc1 = jnp.zeros((16,), jnp.float32)
                for corner in range(8):
                    f0 = rows0_v[pl.ds(corner * _B + s, 16)]
                    f1 = rows1_v[pl.ds(corner * _B + s, 16)]
                    w = w_v[pl.ds(corner * _B + s, 16)]
                    acc0 = acc0 + w * f0
                    acc1 = acc1 + w * f1
                oidx = (s + lane) * 32 + (2 * level)
                plsc.store_scatter(out_v, [oidx], acc0)
                plsc.store_scatter(out_v, [oidx + 1], acc1)
                return 0

            return red_body

        def run_red(level):
            body = make_red_body(level)

            def _b(g):
                body(g, 0)

            plsc.parallel_loop(0, _G, 1, unroll=2)(_b)

        run_idx(0)
        cp = issue(0)
        for level in range(_N_LEVELS):
            if level + 1 < _N_LEVELS:
                run_idx(level + 1)
                cp_next = issue(level + 1)
            cp[0].wait()
            cp[1].wait()
            run_red(level)
            if level + 1 < _N_LEVELS:
                cp = cp_next

        pltpu.sync_copy(out_v, out_hbm.at[pl.ds(base * 32, _B * 32)])
        return 0

    lax.fori_loop(0, nchunks, chunk_body, 0, unroll=False)


def kernel(positions, hash_table):
    n = positions.shape[0]
    px = positions[:, 0]
    py = positions[:, 1]
    pz = positions[:, 2]

    mesh = plsc.VectorSubcoreMesh(core_axis_name="c", subcore_axis_name="s")
    run = functools.partial(
        pl.kernel,
        mesh=mesh,
        compiler_params=pltpu.CompilerParams(needs_layout_passes=False),
        out_type=jax.ShapeDtypeStruct((n * 32,), jnp.float32),
        scratch_types=[
            pltpu.VMEM((_B,), jnp.float32),
            pltpu.VMEM((_B,), jnp.float32),
            pltpu.VMEM((_B,), jnp.float32),
            pltpu.VMEM((8 * _B,), jnp.int32),
            pltpu.VMEM((8 * _B,), jnp.int32),
            pltpu.VMEM((8 * _B,), jnp.int32),
            pltpu.VMEM((8 * _B,), jnp.int32),
            pltpu.VMEM((8 * _B,), jnp.float32),
            pltpu.VMEM((8 * _B,), jnp.float32),
            pltpu.VMEM((8 * _B,), jnp.float32),
            pltpu.VMEM((8 * _B,), jnp.float32),
            pltpu.VMEM((8 * _B,), jnp.float32),
            pltpu.VMEM((8 * _B,), jnp.float32),
            pltpu.VMEM((_B * 32,), jnp.float32),
            pltpu.VMEM_SHARED((_DENSE_PAD,), jnp.float32),
            pltpu.SemaphoreType.DMA,
            pltpu.SemaphoreType.DMA,
        ],
    )(_encode_body)
    out = run(px, py, pz, hash_table)
    return out.reshape(n, 32)
